# Initial kernel scaffold; baseline (speedup 1.0000x reference)
import jax
import jax.numpy as jnp
from jax.experimental import pallas as pl


def kernel(x, indices, emb_weight):
    raise NotImplementedError("write your pallas kernel here")


# speedup vs baseline: 1.0177x; 1.0177x over previous
import jax
import jax.numpy as jnp
from jax.experimental import pallas as pl

_SEQ_BLOCK = 512


def _body(idx_ref, emb_ref, x_ref, o_ref):
    scale = (idx_ref[0, 0, :] - 1).astype(jnp.float32)
    o_ref[0] = x_ref[0] + scale[:, None] * emb_ref[0][None, :]


def kernel(x, indices, emb_weight):
    B, S, D = x.shape
    ns = S // _SEQ_BLOCK
    idx3 = indices.reshape(ns, 1, _SEQ_BLOCK)
    return pl.pallas_call(
        _body,
        grid=(B, ns),
        in_specs=[
            pl.BlockSpec((1, 1, _SEQ_BLOCK), lambda b, s: (s, 0, 0)),
            pl.BlockSpec((1, D), lambda b, s: (0, 0)),
            pl.BlockSpec((1, _SEQ_BLOCK, D), lambda b, s: (b, s, 0)),
        ],
        out_specs=pl.BlockSpec((1, _SEQ_BLOCK, D), lambda b, s: (b, s, 0)),
        out_shape=jax.ShapeDtypeStruct((B, S, D), x.dtype),
    )(idx3, emb_weight, x)


# speedup vs baseline: 1.1410x; 1.1211x over previous
import jax
import jax.numpy as jnp
from jax.experimental import pallas as pl

_SEQ_BLOCK = 1024


def _body(idx_ref, emb_ref, x_ref, o_ref):
    scale = (idx_ref[0, 0, :] - 1).astype(jnp.float32)
    o_ref[0] = x_ref[0] + scale[:, None] * emb_ref[0][None, :]


def kernel(x, indices, emb_weight):
    B, S, D = x.shape
    ns = S // _SEQ_BLOCK
    idx3 = indices.reshape(ns, 1, _SEQ_BLOCK)
    return pl.pallas_call(
        _body,
        grid=(B, ns),
        in_specs=[
            pl.BlockSpec((1, 1, _SEQ_BLOCK), lambda b, s: (s, 0, 0)),
            pl.BlockSpec((1, D), lambda b, s: (0, 0)),
            pl.BlockSpec((1, _SEQ_BLOCK, D), lambda b, s: (b, s, 0)),
        ],
        out_specs=pl.BlockSpec((1, _SEQ_BLOCK, D), lambda b, s: (b, s, 0)),
        out_shape=jax.ShapeDtypeStruct((B, S, D), x.dtype),
    )(idx3, emb_weight, x)


# speedup vs baseline: 1.2278x; 1.0760x over previous
import jax
import jax.numpy as jnp
from jax.experimental import pallas as pl

_SEQ_BLOCK = 2048


def _body(idx_ref, emb_ref, x_ref, o_ref):
    scale = (idx_ref[0, 0, :] - 1).astype(jnp.float32)
    o_ref[0] = x_ref[0] + scale[:, None] * emb_ref[0][None, :]


def kernel(x, indices, emb_weight):
    B, S, D = x.shape
    ns = S // _SEQ_BLOCK
    idx3 = indices.reshape(ns, 1, _SEQ_BLOCK)
    return pl.pallas_call(
        _body,
        grid=(B, ns),
        in_specs=[
            pl.BlockSpec((1, 1, _SEQ_BLOCK), lambda b, s: (s, 0, 0)),
            pl.BlockSpec((1, D), lambda b, s: (0, 0)),
            pl.BlockSpec((1, _SEQ_BLOCK, D), lambda b, s: (b, s, 0)),
        ],
        out_specs=pl.BlockSpec((1, _SEQ_BLOCK, D), lambda b, s: (b, s, 0)),
        out_shape=jax.ShapeDtypeStruct((B, S, D), x.dtype),
    )(idx3, emb_weight, x)
